# baseline (speedup 1.0000x reference)
import functools

import jax
import jax.numpy as jnp
from jax import lax
from jax.experimental import pallas as pl
from jax.experimental.pallas import tpu as pltpu
from jax.experimental.pallas import tpu_sc as plsc

_N = 200000
_D = 256
_B = 32
_NC = 2
_NS = 16
_NW = _NC * _NS
_C = 64
_CD = _C * _D
_QUOTA = 6272
_LAST = _N - (_NW - 1) * _QUOTA
_FULLCH = _QUOTA // _C
_LASTCH = _LAST // _C


def _softmax_body(v_ref, o_ref):
    v = v_ref[...]
    v = v - jnp.max(v, axis=1, keepdims=True)
    e = jnp.exp(v)
    o_ref[...] = e / jnp.sum(e, axis=1, keepdims=True)


def _tc_softmax(vecter):
    return pl.pallas_call(
        _softmax_body,
        out_shape=jax.ShapeDtypeStruct((_B, _D), jnp.float32),
    )(vecter)


def _sc_body(feat_hbm, idx_hbm, vsm_hbm, out_hbm,
             idx_v, feat_v, rows_v, out_v, isem, fsem, gsem, wsem):
    cid = lax.axis_index("c")
    sid = lax.axis_index("s")
    wid = sid * _NC + cid
    nchunks = jnp.where(wid < _NW - 1, _FULLCH, _LASTCH)
    start = wid * _QUOTA

    pltpu.async_copy(idx_hbm.at[pl.ds(start, _QUOTA)], idx_v, isem)
    pltpu.make_async_copy(idx_hbm.at[pl.ds(0, _QUOTA)], idx_v, isem).wait()

    def issue(k, buf):
        base = start + k * _C
        pltpu.async_copy(feat_hbm.at[pl.ds(base * _D, _CD)],
                         feat_v.at[pl.ds(buf * _CD, _CD)], fsem)
        pltpu.async_copy(vsm_hbm.at[idx_v.at[pl.ds(k * _C, _C)]],
                         rows_v.at[buf], gsem)

    issue(0, 0)

    def chunk_body(k, _):
        b = lax.rem(k, 2)
        nb = 1 - b

        @pl.when(k + 1 < nchunks)
        def _():
            issue(k + 1, nb)

        pltpu.make_async_copy(feat_hbm.at[pl.ds(0, _CD)],
                              feat_v.at[pl.ds(b * _CD, _CD)], fsem).wait()
        pltpu.make_async_copy(vsm_hbm.at[idx_v.at[pl.ds(0, _C)]],
                              rows_v.at[b], gsem).wait()

        @pl.when(k >= 2)
        def _():
            pltpu.make_async_copy(out_v.at[pl.ds(b * _CD, _CD)],
                                  out_hbm.at[pl.ds(0, _CD)], wsem).wait()

        cbase = b * _CD

        @plsc.parallel_loop(0, _C, unroll=2)
        def _row(r):
            rbase = cbase + r * _D
            for j in range(_D // 16):
                a = feat_v[pl.ds(rbase + j * 16, 16)]
                t = rows_v[b, r, pl.ds(j * 16, 16)]
                out_v[pl.ds(rbase + j * 16, 16)] = a * t

        base = start + k * _C
        pltpu.async_copy(out_v.at[pl.ds(cbase, _CD)],
                         out_hbm.at[pl.ds(base * _D, _CD)], wsem)
        return 0

    lax.fori_loop(0, nchunks, chunk_body, 0)
    pltpu.make_async_copy(out_v.at[pl.ds(0, _CD)],
                          out_hbm.at[pl.ds(0, _CD)], wsem).wait()
    pltpu.make_async_copy(out_v.at[pl.ds(_CD, _CD)],
                          out_hbm.at[pl.ds(0, _CD)], wsem).wait()


def kernel(features, point_idx, vecter):
    vsm = _tc_softmax(vecter)
    feat_flat = features.reshape(_N * _D)
    idx = jnp.pad(point_idx.astype(jnp.int32), (0, _NW * _QUOTA - _N))

    mesh = plsc.VectorSubcoreMesh(core_axis_name="c", subcore_axis_name="s")
    out_flat = pl.kernel(
        _sc_body,
        out_type=jax.ShapeDtypeStruct((_N * _D,), jnp.float32),
        mesh=mesh,
        scratch_types=[
            pltpu.VMEM((_QUOTA,), jnp.int32),
            pltpu.VMEM((2 * _CD,), jnp.float32),
            pltpu.VMEM((2, _C, _D), jnp.float32),
            pltpu.VMEM((2 * _CD,), jnp.float32),
            pltpu.SemaphoreType.DMA,
            pltpu.SemaphoreType.DMA,
            pltpu.SemaphoreType.DMA,
            pltpu.SemaphoreType.DMA,
        ],
    )(feat_flat, idx, vsm)
    return out_flat.reshape(_N, _D)


# speedup vs baseline: 1.7018x; 1.7018x over previous
import functools

import jax
import jax.numpy as jnp
from jax import lax
from jax.experimental import pallas as pl
from jax.experimental.pallas import tpu as pltpu
from jax.experimental.pallas import tpu_sc as plsc

_N = 200000
_D = 256
_B = 32
_NC = 2
_NS = 16
_NW = _NC * _NS
_C = 64
_CD = _C * _D
_NSC = 64000
_QUOTA = 2048
_LAST = _NSC - (_NW - 1) * _QUOTA
_FULLCH = _QUOTA // _C
_LASTCH = _LAST // _C


def _softmax_body(v_ref, o_ref):
    v = v_ref[...]
    v = v - jnp.max(v, axis=1, keepdims=True)
    e = jnp.exp(v)
    o_ref[...] = e / jnp.sum(e, axis=1, keepdims=True)


def _tc_softmax(vecter):
    return pl.pallas_call(
        _softmax_body,
        out_shape=jax.ShapeDtypeStruct((_B, _D), jnp.float32),
    )(vecter)


def _sc_body(feat_hbm, idx_hbm, vsm_hbm, out_hbm,
             idx_v, feat_v, rows_v, out_v, isem, fsem, gsem, wsem):
    cid = lax.axis_index("c")
    sid = lax.axis_index("s")
    wid = sid * _NC + cid
    nchunks = jnp.where(wid < _NW - 1, _FULLCH, _LASTCH)
    start = wid * _QUOTA

    pltpu.async_copy(idx_hbm.at[pl.ds(start, _QUOTA)], idx_v, isem)
    pltpu.make_async_copy(idx_hbm.at[pl.ds(0, _QUOTA)], idx_v, isem).wait()

    def issue(k, buf):
        base = start + k * _C
        pltpu.async_copy(feat_hbm.at[pl.ds(base * _D, _CD)],
                         feat_v.at[pl.ds(buf * _CD, _CD)], fsem)
        pltpu.async_copy(vsm_hbm.at[idx_v.at[pl.ds(k * _C, _C)]],
                         rows_v.at[buf], gsem)

    issue(0, 0)

    def chunk_body(k, _):
        b = lax.rem(k, 2)
        nb = 1 - b

        @pl.when(k + 1 < nchunks)
        def _():
            issue(k + 1, nb)

        pltpu.make_async_copy(feat_hbm.at[pl.ds(0, _CD)],
                              feat_v.at[pl.ds(b * _CD, _CD)], fsem).wait()
        pltpu.make_async_copy(vsm_hbm.at[idx_v.at[pl.ds(0, _C)]],
                              rows_v.at[b], gsem).wait()

        @pl.when(k >= 2)
        def _():
            pltpu.make_async_copy(out_v.at[pl.ds(b * _CD, _CD)],
                                  out_hbm.at[pl.ds(0, _CD)], wsem).wait()

        cbase = b * _CD

        @plsc.parallel_loop(0, _C, unroll=2)
        def _row(r):
            rbase = cbase + r * _D
            for j in range(_D // 16):
                a = feat_v[pl.ds(rbase + j * 16, 16)]
                t = rows_v[b, r, pl.ds(j * 16, 16)]
                out_v[pl.ds(rbase + j * 16, 16)] = a * t

        base = start + k * _C
        pltpu.async_copy(out_v.at[pl.ds(cbase, _CD)],
                         out_hbm.at[pl.ds(base * _D, _CD)], wsem)
        return 0

    lax.fori_loop(0, nchunks, chunk_body, 0)
    pltpu.make_async_copy(out_v.at[pl.ds(0, _CD)],
                          out_hbm.at[pl.ds(0, _CD)], wsem).wait()
    pltpu.make_async_copy(out_v.at[pl.ds(_CD, _CD)],
                          out_hbm.at[pl.ds(0, _CD)], wsem).wait()


_BR = 4000
_NTC = _N - _NSC
_TCG = _NTC // _BR
_OFF = _NSC // _BR


def _tc_body(idx_ref, feat_ref, vsm_ref, out_ref):
    idx = idx_ref[0]
    rows = jax.lax.broadcasted_iota(jnp.int32, (_B, _BR), 0)
    onehot_t = jnp.where(idx == rows, 1.0, 0.0).astype(jnp.float32)
    gathered = jax.lax.dot_general(
        onehot_t, vsm_ref[...], (((0,), (0,)), ((), ())),
        preferred_element_type=jnp.float32)
    out_ref[...] = feat_ref[...] * gathered


def _tc_part(features, idx3d, vsm):
    return pl.pallas_call(
        _tc_body,
        grid=(_TCG,),
        in_specs=[
            pl.BlockSpec((1, 1, _BR), lambda i: (i + _OFF, 0, 0)),
            pl.BlockSpec((_BR, _D), lambda i: (i + _OFF, 0)),
            pl.BlockSpec((_B, _D), lambda i: (0, 0)),
        ],
        out_specs=pl.BlockSpec((_BR, _D), lambda i: (i, 0)),
        out_shape=jax.ShapeDtypeStruct((_NTC, _D), jnp.float32),
    )(idx3d, features, vsm)


def kernel(features, point_idx, vecter):
    vsm = _tc_softmax(vecter)
    feat_flat = features.reshape(_N * _D)
    idx = point_idx.astype(jnp.int32)
    idx3d = idx.reshape(_N // _BR, 1, _BR)

    mesh = plsc.VectorSubcoreMesh(core_axis_name="c", subcore_axis_name="s")
    sc_flat = pl.kernel(
        _sc_body,
        out_type=jax.ShapeDtypeStruct((_NSC * _D,), jnp.float32),
        mesh=mesh,
        scratch_types=[
            pltpu.VMEM((_QUOTA,), jnp.int32),
            pltpu.VMEM((2 * _CD,), jnp.float32),
            pltpu.VMEM((2, _C, _D), jnp.float32),
            pltpu.VMEM((2 * _CD,), jnp.float32),
            pltpu.SemaphoreType.DMA,
            pltpu.SemaphoreType.DMA,
            pltpu.SemaphoreType.DMA,
            pltpu.SemaphoreType.DMA,
        ],
    )(feat_flat, idx, vsm)
    tc_out = _tc_part(features, idx3d, vsm)
    return jnp.concatenate([sc_flat.reshape(_NSC, _D), tc_out], axis=0)


# speedup vs baseline: 1.9288x; 1.1334x over previous
import functools

import jax
import jax.numpy as jnp
from jax import lax
from jax.experimental import pallas as pl
from jax.experimental.pallas import tpu as pltpu
from jax.experimental.pallas import tpu_sc as plsc

_N = 200000
_D = 256
_B = 32
_NC = 2
_NS = 16
_NW = _NC * _NS
_C = 64
_CD = _C * _D
_QUOTA = 6272
_LAST = _N - (_NW - 1) * _QUOTA
_FULLCH = _QUOTA // _C
_LASTCH = _LAST // _C


def _softmax_body(v_ref, o_ref):
    v = v_ref[...]
    v = v - jnp.max(v, axis=1, keepdims=True)
    e = jnp.exp(v)
    o_ref[...] = e / jnp.sum(e, axis=1, keepdims=True)


def _tc_softmax(vecter):
    return pl.pallas_call(
        _softmax_body,
        out_shape=jax.ShapeDtypeStruct((_B, _D), jnp.float32),
    )(vecter)


def _sc_body(feat_hbm, idx_hbm, vsm_hbm, out_hbm,
             idx_v, feat_v, rows_v, out_v, isem, fsem, gsem, wsem):
    cid = lax.axis_index("c")
    sid = lax.axis_index("s")
    wid = sid * _NC + cid
    nchunks = jnp.where(wid < _NW - 1, _FULLCH, _LASTCH)
    start = wid * _QUOTA

    pltpu.async_copy(idx_hbm.at[pl.ds(start, _QUOTA)], idx_v, isem)
    pltpu.make_async_copy(idx_hbm.at[pl.ds(0, _QUOTA)], idx_v, isem).wait()

    woff = wid * _B

    @plsc.parallel_loop(0, _QUOTA // 16, unroll=4)
    def _adj(t):
        idx_v[pl.ds(t * 16, 16)] = idx_v[pl.ds(t * 16, 16)] + woff

    def issue(k, buf):
        base = start + k * _C
        pltpu.async_copy(feat_hbm.at[pl.ds(base * _D, _CD)],
                         feat_v.at[pl.ds(buf * _CD, _CD)], fsem)
        pltpu.async_copy(vsm_hbm.at[idx_v.at[pl.ds(k * _C, _C)]],
                         rows_v.at[buf], gsem)

    issue(0, 0)

    def chunk_body(k, _):
        b = lax.rem(k, 2)
        nb = 1 - b

        @pl.when(k + 1 < nchunks)
        def _():
            issue(k + 1, nb)

        pltpu.make_async_copy(feat_hbm.at[pl.ds(0, _CD)],
                              feat_v.at[pl.ds(b * _CD, _CD)], fsem).wait()
        pltpu.make_async_copy(vsm_hbm.at[idx_v.at[pl.ds(0, _C)]],
                              rows_v.at[b], gsem).wait()

        @pl.when(k >= 2)
        def _():
            pltpu.make_async_copy(out_v.at[pl.ds(b * _CD, _CD)],
                                  out_hbm.at[pl.ds(0, _CD)], wsem).wait()

        cbase = b * _CD

        @plsc.parallel_loop(0, _C, unroll=2)
        def _row(r):
            rbase = cbase + r * _D
            for j in range(_D // 16):
                a = feat_v[pl.ds(rbase + j * 16, 16)]
                t = rows_v[b, r, pl.ds(j * 16, 16)]
                out_v[pl.ds(rbase + j * 16, 16)] = a * t

        base = start + k * _C
        pltpu.async_copy(out_v.at[pl.ds(cbase, _CD)],
                         out_hbm.at[pl.ds(base * _D, _CD)], wsem)
        return 0

    lax.fori_loop(0, nchunks, chunk_body, 0)
    pltpu.make_async_copy(out_v.at[pl.ds(0, _CD)],
                          out_hbm.at[pl.ds(0, _CD)], wsem).wait()
    pltpu.make_async_copy(out_v.at[pl.ds(_CD, _CD)],
                          out_hbm.at[pl.ds(0, _CD)], wsem).wait()


def kernel(features, point_idx, vecter):
    vsm = _tc_softmax(vecter)
    feat_flat = features.reshape(_N * _D)
    idx = jnp.pad(point_idx.astype(jnp.int32), (0, _NW * _QUOTA - _N))
    vsm_rep = jnp.tile(vsm, (_NW, 1))

    mesh = plsc.VectorSubcoreMesh(core_axis_name="c", subcore_axis_name="s")
    out_flat = pl.kernel(
        _sc_body,
        out_type=jax.ShapeDtypeStruct((_N * _D,), jnp.float32),
        mesh=mesh,
        scratch_types=[
            pltpu.VMEM((_QUOTA,), jnp.int32),
            pltpu.VMEM((2 * _CD,), jnp.float32),
            pltpu.VMEM((2, _C, _D), jnp.float32),
            pltpu.VMEM((2 * _CD,), jnp.float32),
            pltpu.SemaphoreType.DMA,
            pltpu.SemaphoreType.DMA,
            pltpu.SemaphoreType.DMA,
            pltpu.SemaphoreType.DMA,
        ],
    )(feat_flat, idx, vsm_rep)
    return out_flat.reshape(_N, _D)


# speedup vs baseline: 1.9711x; 1.0219x over previous
import functools

import jax
import jax.numpy as jnp
from jax import lax
from jax.experimental import pallas as pl
from jax.experimental.pallas import tpu as pltpu
from jax.experimental.pallas import tpu_sc as plsc

_N = 200000
_D = 256
_B = 32
_NC = 2
_NS = 16
_NW = _NC * _NS
_C = 64
_CD = _C * _D
_QUOTA = 6272
_LAST = _N - (_NW - 1) * _QUOTA
_FULLCH = _QUOTA // _C
_LASTCH = _LAST // _C


def _softmax_body(v_ref, o_ref):
    v = v_ref[...]
    v = v - jnp.max(v, axis=1, keepdims=True)
    e = jnp.exp(v)
    o_ref[...] = e / jnp.sum(e, axis=1, keepdims=True)


def _tc_softmax_tiled(vecter):
    return pl.pallas_call(
        _softmax_body,
        grid=(_NW,),
        in_specs=[pl.BlockSpec((_B, _D), lambda i: (0, 0))],
        out_specs=pl.BlockSpec((_B, _D), lambda i: (i, 0)),
        out_shape=jax.ShapeDtypeStruct((_NW * _B, _D), jnp.float32),
    )(vecter)


def _sc_body(feat_hbm, idx_hbm, vsm_hbm, out_hbm,
             idx_v, feat_v, rows_v, out_v, isem, fsem, gsem, wsem):
    cid = lax.axis_index("c")
    sid = lax.axis_index("s")
    wid = sid * _NC + cid
    nchunks = jnp.where(wid < _NW - 1, _FULLCH, _LASTCH)
    start = wid * _QUOTA

    @pl.when(wid < _NW - 1)
    def _():
        pltpu.async_copy(idx_hbm.at[pl.ds(start, _QUOTA)], idx_v, isem)
        pltpu.make_async_copy(idx_hbm.at[pl.ds(0, _QUOTA)], idx_v, isem).wait()

    @pl.when(wid == _NW - 1)
    def _():
        pltpu.async_copy(idx_hbm.at[pl.ds(start, _LAST)],
                         idx_v.at[pl.ds(0, _LAST)], isem)
        pltpu.make_async_copy(idx_hbm.at[pl.ds(0, _LAST)],
                              idx_v.at[pl.ds(0, _LAST)], isem).wait()

    woff = wid * _B

    @plsc.parallel_loop(0, _QUOTA // 16, unroll=4)
    def _adj(t):
        idx_v[pl.ds(t * 16, 16)] = idx_v[pl.ds(t * 16, 16)] + woff

    def issue(k, buf):
        base = start + k * _C
        pltpu.async_copy(feat_hbm.at[pl.ds(base * _D, _CD)],
                         feat_v.at[pl.ds(buf * _CD, _CD)], fsem)
        pltpu.async_copy(vsm_hbm.at[idx_v.at[pl.ds(k * _C, _C)]],
                         rows_v.at[buf], gsem)

    issue(0, 0)

    def chunk_body(k, _):
        b = lax.rem(k, 2)
        nb = 1 - b

        @pl.when(k + 1 < nchunks)
        def _():
            issue(k + 1, nb)

        pltpu.make_async_copy(feat_hbm.at[pl.ds(0, _CD)],
                              feat_v.at[pl.ds(b * _CD, _CD)], fsem).wait()
        pltpu.make_async_copy(vsm_hbm.at[idx_v.at[pl.ds(0, _C)]],
                              rows_v.at[b], gsem).wait()

        @pl.when(k >= 2)
        def _():
            pltpu.make_async_copy(out_v.at[pl.ds(b * _CD, _CD)],
                                  out_hbm.at[pl.ds(0, _CD)], wsem).wait()

        cbase = b * _CD

        @plsc.parallel_loop(0, _C, unroll=2)
        def _row(r):
            rbase = cbase + r * _D
            for j in range(_D // 16):
                a = feat_v[pl.ds(rbase + j * 16, 16)]
                t = rows_v[b, r, pl.ds(j * 16, 16)]
                out_v[pl.ds(rbase + j * 16, 16)] = a * t

        base = start + k * _C
        pltpu.async_copy(out_v.at[pl.ds(cbase, _CD)],
                         out_hbm.at[pl.ds(base * _D, _CD)], wsem)
        return 0

    lax.fori_loop(0, nchunks, chunk_body, 0)
    pltpu.make_async_copy(out_v.at[pl.ds(0, _CD)],
                          out_hbm.at[pl.ds(0, _CD)], wsem).wait()
    pltpu.make_async_copy(out_v.at[pl.ds(_CD, _CD)],
                          out_hbm.at[pl.ds(0, _CD)], wsem).wait()


def kernel(features, point_idx, vecter):
    vsm_rep = _tc_softmax_tiled(vecter)
    feat_flat = features.reshape(_N * _D)
    idx = point_idx.astype(jnp.int32)

    mesh = plsc.VectorSubcoreMesh(core_axis_name="c", subcore_axis_name="s")
    out_flat = pl.kernel(
        _sc_body,
        out_type=jax.ShapeDtypeStruct((_N * _D,), jnp.float32),
        mesh=mesh,
        scratch_types=[
            pltpu.VMEM((_QUOTA,), jnp.int32),
            pltpu.VMEM((2 * _CD,), jnp.float32),
            pltpu.VMEM((2, _C, _D), jnp.float32),
            pltpu.VMEM((2 * _CD,), jnp.float32),
            pltpu.SemaphoreType.DMA,
            pltpu.SemaphoreType.DMA,
            pltpu.SemaphoreType.DMA,
            pltpu.SemaphoreType.DMA,
        ],
    )(feat_flat, idx, vsm_rep)
    return out_flat.reshape(_N, _D)
